# Initial kernel scaffold; baseline (speedup 1.0000x reference)
import jax
import jax.numpy as jnp
from jax.experimental import pallas as pl


def kernel(X, edge_index, edge_weight, W1, b1, W2, b2, W3, b3, Wmu, bmu, Wls, bls):
    raise NotImplementedError("write your pallas kernel here")


# speedup vs baseline: 7.8715x; 7.8715x over previous
import functools

import jax
import jax.numpy as jnp
from jax import lax
from jax.experimental import pallas as pl
from jax.experimental.pallas import tpu as pltpu
from jax.experimental.pallas import tpu_sc as plsc

N = 10000
E = 320000
NPD = 10240
NPT = 640
NC, NS = 2, 16
NT = 624
NREM = N - NS * NT
NW = NC * NS
WIN = 128
NWIN = 79
EPW = NWIN * WIN
E_PAD = NW * EPW
CH = 128
R = 1000

_mesh = plsc.VectorSubcoreMesh(core_axis_name="c", subcore_axis_name="s")
_sc_params = pltpu.CompilerParams(needs_layout_passes=False)


def _zero_2d(buf, rows):
    z = jnp.zeros((16,), jnp.float32)

    def body(r, carry):
        for j in range(8):
            buf[r, pl.ds(16 * j, 16)] = z
        return carry

    lax.fori_loop(0, rows, body, 0)


def _zero_1d(buf, chunks):
    z = jnp.zeros((16,), jnp.float32)

    def body(i, carry):
        buf[pl.ds(16 * i, 16)] = z
        return carry

    lax.fori_loop(0, chunks, body, 0)


def _zero_acc(s, zbuf, acc_sh):
    _zero_2d(zbuf, 16)
    base = s * NT
    for k in range(NT // 16):
        pltpu.sync_copy(zbuf, acc_sh.at[pl.ds(base + 16 * k, 16)])

    @pl.when(s == NS - 1)
    def _():
        pltpu.sync_copy(zbuf, acc_sh.at[pl.ds(NS * NT, NREM)])


def _acc_out(c, s, acc_sh, g_hbm):
    base = s * NT
    pltpu.sync_copy(acc_sh.at[pl.ds(base, NT)], g_hbm.at[c, pl.ds(base, NT)])

    @pl.when(s == NS - 1)
    def _():
        pltpu.sync_copy(acc_sh.at[pl.ds(NS * NT, NREM)],
                        g_hbm.at[c, pl.ds(NS * NT, NREM)])


def _scale_and_scatter(w, row_v, col_v, norm_v, gbuf, h_hbm, acc_sh, sem):
    pltpu.async_copy(h_hbm.at[row_v.at[w]], gbuf, sem).wait()
    w16 = jnp.full((16,), w, jnp.int32)

    def srow(r, carry):
        nb = plsc.load_gather(norm_v, [w16, jnp.full((16,), r, jnp.int32)])
        for j in range(8):
            sl = pl.ds(16 * j, 16)
            gbuf[r, sl] = gbuf[r, sl] * nb
        return carry

    lax.fori_loop(0, WIN, srow, 0)
    pltpu.sync_copy(gbuf, acc_sh.at[col_v.at[w]], add=True)


@functools.partial(
    pl.kernel,
    out_type=jax.ShapeDtypeStruct((NC, NPD), jnp.float32),
    mesh=_mesh,
    compiler_params=_sc_params,
    scratch_types=[
        pltpu.VMEM((NWIN, WIN), jnp.int32),
        pltpu.VMEM((NWIN, WIN), jnp.float32),
        pltpu.VMEM((NPT,), jnp.float32),
        pltpu.VMEM_SHARED((NPD,), jnp.float32),
    ],
)
def _sc_deg(col_hbm, we_hbm, deg_hbm, col_v, w_v, dbuf, deg_sh):
    c = lax.axis_index("c")
    s = lax.axis_index("s")
    wid = c * NS + s
    base = s * NPT

    _zero_1d(dbuf, NPT // 16)
    pltpu.sync_copy(dbuf, deg_sh.at[pl.ds(base, NPT)])
    pltpu.sync_copy(col_hbm.at[wid], col_v)
    pltpu.sync_copy(we_hbm.at[wid], w_v)
    plsc.subcore_barrier()

    def dscat(i, carry):
        pltpu.sync_copy(w_v.at[i], deg_sh.at[col_v.at[i]], add=True)
        return carry

    lax.fori_loop(0, NWIN, dscat, 0)
    plsc.subcore_barrier()
    pltpu.sync_copy(deg_sh.at[pl.ds(base, NPT)],
                    deg_hbm.at[c, pl.ds(base, NPT)])


@functools.partial(
    pl.kernel,
    out_type=(
        jax.ShapeDtypeStruct((NC, N, CH), jnp.float32),
        jax.ShapeDtypeStruct((NW, NWIN, WIN), jnp.float32),
    ),
    mesh=_mesh,
    compiler_params=_sc_params,
    scratch_types=[
        pltpu.VMEM((NWIN, WIN), jnp.int32),
        pltpu.VMEM((NWIN, WIN), jnp.int32),
        pltpu.VMEM((NWIN, WIN), jnp.float32),
        pltpu.VMEM((WIN,), jnp.float32),
        pltpu.VMEM((WIN,), jnp.float32),
        pltpu.VMEM((WIN, CH), jnp.float32),
        pltpu.VMEM((16, CH), jnp.float32),
        pltpu.VMEM_SHARED((N, CH), jnp.float32),
        pltpu.SemaphoreType.DMA,
    ],
)
def _sc_norm_agg(row_hbm, col_hbm, we_hbm, dis_hbm, h_hbm, g_hbm, norm_hbm,
                 row_v, col_v, norm_v, drbuf, dcbuf, gbuf, zbuf, acc_sh, sem):
    c = lax.axis_index("c")
    s = lax.axis_index("s")
    wid = c * NS + s

    _zero_acc(s, zbuf, acc_sh)
    pltpu.sync_copy(row_hbm.at[wid], row_v)
    pltpu.sync_copy(col_hbm.at[wid], col_v)
    pltpu.sync_copy(we_hbm.at[wid], norm_v)
    plsc.subcore_barrier()

    def nwin(w, carry):
        pltpu.async_copy(dis_hbm.at[row_v.at[w]], drbuf, sem).wait()
        pltpu.async_copy(dis_hbm.at[col_v.at[w]], dcbuf, sem).wait()
        for k in range(8):
            sl = pl.ds(16 * k, 16)
            norm_v[w, sl] = drbuf[sl] * norm_v[w, sl] * dcbuf[sl]
        return carry

    lax.fori_loop(0, NWIN, nwin, 0)
    pltpu.sync_copy(norm_v, norm_hbm.at[wid])

    def awin(w, carry):
        _scale_and_scatter(w, row_v, col_v, norm_v, gbuf, h_hbm, acc_sh, sem)
        return carry

    lax.fori_loop(0, NWIN, awin, 0)
    plsc.subcore_barrier()
    _acc_out(c, s, acc_sh, g_hbm)


@functools.partial(
    pl.kernel,
    out_type=jax.ShapeDtypeStruct((NC, N, CH), jnp.float32),
    mesh=_mesh,
    compiler_params=_sc_params,
    scratch_types=[
        pltpu.VMEM((NWIN, WIN), jnp.int32),
        pltpu.VMEM((NWIN, WIN), jnp.int32),
        pltpu.VMEM((NWIN, WIN), jnp.float32),
        pltpu.VMEM((WIN, CH), jnp.float32),
        pltpu.VMEM((16, CH), jnp.float32),
        pltpu.VMEM_SHARED((N, CH), jnp.float32),
        pltpu.SemaphoreType.DMA,
    ],
)
def _sc_agg(row_hbm, col_hbm, norm_hbm, h_hbm, g_hbm,
            row_v, col_v, norm_v, gbuf, zbuf, acc_sh, sem):
    c = lax.axis_index("c")
    s = lax.axis_index("s")
    wid = c * NS + s

    _zero_acc(s, zbuf, acc_sh)
    pltpu.sync_copy(row_hbm.at[wid], row_v)
    pltpu.sync_copy(col_hbm.at[wid], col_v)
    pltpu.sync_copy(norm_hbm.at[wid], norm_v)
    plsc.subcore_barrier()

    def awin(w, carry):
        _scale_and_scatter(w, row_v, col_v, norm_v, gbuf, h_hbm, acc_sh, sem)
        return carry

    lax.fori_loop(0, NWIN, awin, 0)
    plsc.subcore_barrier()
    _acc_out(c, s, acc_sh, g_hbm)


def _prep_body(degp_ref, x_ref, w_ref, dis_ref, inv_ref, h_ref):
    deg = 1.0 + degp_ref[0] + degp_ref[1]
    dis_ref[...] = lax.rsqrt(deg)
    inv_ref[...] = 1.0 / deg
    h_ref[...] = jnp.dot(x_ref[...], w_ref[...],
                         preferred_element_type=jnp.float32)


def _tc_prep(degp3, x, w):
    return pl.pallas_call(
        _prep_body,
        grid=(N // R,),
        in_specs=[pl.BlockSpec((NC, R, 1), lambda i: (0, i, 0)),
                  pl.BlockSpec((R, CH), lambda i: (i, 0)),
                  pl.BlockSpec((CH, CH), lambda i: (0, 0))],
        out_specs=[pl.BlockSpec((R, 1), lambda i: (i, 0)),
                   pl.BlockSpec((R, 1), lambda i: (i, 0)),
                   pl.BlockSpec((R, CH), lambda i: (i, 0))],
        out_shape=[jax.ShapeDtypeStruct((N, 1), jnp.float32),
                   jax.ShapeDtypeStruct((N, 1), jnp.float32),
                   jax.ShapeDtypeStruct((N, CH), jnp.float32)],
    )(degp3, x, w)


def _comb_body(g_ref, h_ref, inv_ref, b_ref, w_ref, o_ref):
    x = jnp.maximum(g_ref[0] + g_ref[1] + inv_ref[...] * h_ref[...]
                    + b_ref[...], 0.0)
    o_ref[...] = jnp.dot(x, w_ref[...], preferred_element_type=jnp.float32)


def _tc_comb_mm(g, h, inv2, b2d, w):
    return pl.pallas_call(
        _comb_body,
        grid=(N // R,),
        in_specs=[pl.BlockSpec((NC, R, CH), lambda i: (0, i, 0)),
                  pl.BlockSpec((R, CH), lambda i: (i, 0)),
                  pl.BlockSpec((R, 1), lambda i: (i, 0)),
                  pl.BlockSpec((1, CH), lambda i: (0, 0)),
                  pl.BlockSpec((CH, CH), lambda i: (0, 0))],
        out_specs=pl.BlockSpec((R, CH), lambda i: (i, 0)),
        out_shape=jax.ShapeDtypeStruct((N, CH), jnp.float32),
    )(g, h, inv2, b2d, w)


def _combx_body(g_ref, h_ref, inv_ref, b_ref, o_ref):
    o_ref[...] = jnp.maximum(g_ref[0] + g_ref[1]
                             + inv_ref[...] * h_ref[...] + b_ref[...], 0.0)


def _tc_comb_x(g, h, inv2, b2d):
    return pl.pallas_call(
        _combx_body,
        grid=(N // R,),
        in_specs=[pl.BlockSpec((NC, R, CH), lambda i: (0, i, 0)),
                  pl.BlockSpec((R, CH), lambda i: (i, 0)),
                  pl.BlockSpec((R, 1), lambda i: (i, 0)),
                  pl.BlockSpec((1, CH), lambda i: (0, 0))],
        out_specs=pl.BlockSpec((R, CH), lambda i: (i, 0)),
        out_shape=jax.ShapeDtypeStruct((N, CH), jnp.float32),
    )(g, h, inv2, b2d)


def _final_body(g_ref, x_ref, inv_ref, wmu_ref, bmu_ref, wls_ref, bls_ref,
                mu_ref, ls_ref):
    y = g_ref[0] + g_ref[1] + inv_ref[...] * x_ref[...]
    mu_ref[...] = jnp.dot(y, wmu_ref[...],
                          preferred_element_type=jnp.float32) + bmu_ref[...]
    ls_ref[...] = jnp.dot(y, wls_ref[...],
                          preferred_element_type=jnp.float32) + bls_ref[...]


def _tc_final(g, x3, inv2, wmu, bmu2, wls, bls2):
    oc = wmu.shape[1]
    return pl.pallas_call(
        _final_body,
        grid=(N // R,),
        in_specs=[pl.BlockSpec((NC, R, CH), lambda i: (0, i, 0)),
                  pl.BlockSpec((R, CH), lambda i: (i, 0)),
                  pl.BlockSpec((R, 1), lambda i: (i, 0)),
                  pl.BlockSpec((CH, oc), lambda i: (0, 0)),
                  pl.BlockSpec((1, oc), lambda i: (0, 0)),
                  pl.BlockSpec((CH, oc), lambda i: (0, 0)),
                  pl.BlockSpec((1, oc), lambda i: (0, 0))],
        out_specs=[pl.BlockSpec((R, oc), lambda i: (i, 0)),
                   pl.BlockSpec((R, oc), lambda i: (i, 0))],
        out_shape=[jax.ShapeDtypeStruct((N, oc), jnp.float32),
                   jax.ShapeDtypeStruct((N, oc), jnp.float32)],
    )(g, x3, inv2, wmu, bmu2, wls, bls2)


def kernel(X, edge_index, edge_weight, W1, b1, W2, b2, W3, b3,
           Wmu, bmu, Wls, bls):
    pad = E_PAD - E
    rowp = jnp.pad(edge_index[0], (0, pad)).reshape(NW, NWIN, WIN)
    colp = jnp.pad(edge_index[1], (0, pad)).reshape(NW, NWIN, WIN)
    wp = jnp.pad(edge_weight, (0, pad)).reshape(NW, NWIN, WIN)
    b1r, b2r, b3r = b1.reshape(1, CH), b2.reshape(1, CH), b3.reshape(1, CH)
    bmur, blsr = bmu.reshape(1, -1), bls.reshape(1, -1)

    degp = _sc_deg(colp, wp)
    dis2, inv2, h1 = _tc_prep(degp[:, :N].reshape(NC, N, 1), X, W1)
    g1, normp = _sc_norm_agg(rowp, colp, wp, dis2.reshape(N), h1)
    h2 = _tc_comb_mm(g1, h1, inv2, b1r, W2)
    g2 = _sc_agg(rowp, colp, normp, h2)
    h3 = _tc_comb_mm(g2, h2, inv2, b2r, W3)
    g3 = _sc_agg(rowp, colp, normp, h3)
    x3 = _tc_comb_x(g3, h3, inv2, b3r)
    g4 = _sc_agg(rowp, colp, normp, x3)
    mu, ls = _tc_final(g4, x3, inv2, Wmu, bmur, Wls, blsr)
    return mu, ls
